# Initial kernel scaffold; baseline (speedup 1.0000x reference)
import jax
import jax.numpy as jnp
from jax.experimental import pallas as pl


def kernel(vertices, faces, eps):
    raise NotImplementedError("write your pallas kernel here")


# speedup vs baseline: 15.5682x; 15.5682x over previous
import functools

import jax
import jax.numpy as jnp
from jax import lax
from jax.experimental import pallas as pl
from jax.experimental.pallas import tpu as pltpu
from jax.experimental.pallas import tpu_sc as plsc

L = 16
NF = 2048
NE = 2 * NF
FLAT = 3 * NF
NV = 256
TBL = NV * NV


def _iota():
    return lax.iota(jnp.int32, L)


def _splat_i(x):
    return jnp.full((L,), x, jnp.int32)


def _rsqrt(v):
    i = plsc.bitcast(v, jnp.int32)
    i = _splat_i(0x5F3759DF) - (i >> 1)
    y = plsc.bitcast(i, jnp.float32)
    for _ in range(3):
        y = y * (1.5 - 0.5 * v * y * y)
    return y


def _sqrt(v):
    return v * _rsqrt(v)


def _splat_at(flat_ref, pos):
    chunk = flat_ref[pl.ds(pos, L)]
    return jnp.full((L,), chunk[0], jnp.int32)


def _first_not_in(flat_ref, xs_v, ys_v):
    def cond(st):
        j, pos = st
        return jnp.logical_and(j < FLAT // L, pos < 0)

    def body(st):
        j, pos = st
        chunk = flat_ref[pl.ds(j * L, L)]
        m = jnp.logical_and(chunk != xs_v, chunk != ys_v)
        c = lax.reduce_min(jnp.where(m, _iota(), _splat_i(L)), axes=(0,))
        pos2 = jnp.where(c < L, j * L + c, -1)
        return j + 1, pos2

    _, pos = lax.while_loop(cond, body, (jnp.int32(0), jnp.int32(-1)))
    pos = jnp.maximum(pos, 0)
    return _splat_at(flat_ref, pos)


def _sc_body(verts_hbm, faces_hbm, eps_hbm, zeros_hbm, out_hbm,
             faces0_v, faces1_v, myfaces_v, verts_v, table_v,
             v0buf, v1buf, pres0_v, pres1_v, eps_v, acc_v):
    cid = lax.axis_index("c")
    sid = lax.axis_index("s")

    @pl.when(sid == 0)
    def _():
        pltpu.sync_copy(faces_hbm.at[0], faces0_v.at[pl.ds(0, FLAT)])
        pltpu.sync_copy(faces_hbm.at[1], faces1_v.at[pl.ds(0, FLAT)])
        pltpu.sync_copy(faces_hbm.at[cid], myfaces_v)
        pltpu.sync_copy(verts_hbm.at[cid], verts_v)
        pltpu.sync_copy(zeros_hbm, table_v)
        pltpu.sync_copy(eps_hbm, eps_v)

        zero_i = _splat_i(0)
        one_i = _splat_i(1)
        for b, pres in ((0, pres0_v), (1, pres1_v)):
            for j in range(NV // L):
                pres[pl.ds(j * L, L)] = zero_i

        for flat_ref, pres in ((faces0_v, pres0_v), (faces1_v, pres1_v)):
            def pres_body(j, _, flat_ref=flat_ref, pres=pres):
                chunk = flat_ref[pl.ds(j * L, L)]
                plsc.store_scatter(pres, [chunk], one_i)
                return 0

            lax.fori_loop(0, FLAT // L, pres_body, 0)

        xyzs = []
        for flat_ref in (faces0_v, faces1_v):
            xs = _splat_at(flat_ref, 0)
            ys = _first_not_in(flat_ref, xs, xs)
            zs = _first_not_in(flat_ref, xs, ys)
            xyzs.append((xs, ys, zs))

        one_f = jnp.full((L,), 1.0, jnp.float32)
        for h in range(2):
            def p1_body(t, _, h=h):
                ii = _iota() + t * L
                base = ii * 3 + h
                a = plsc.load_gather(myfaces_v, [base])
                b = plsc.load_gather(myfaces_v, [base + 1])
                v0 = jnp.minimum(a, b)
                v1 = jnp.maximum(a, b)
                v0buf[pl.ds(h * NF + t * L, L)] = v0
                v1buf[pl.ds(h * NF + t * L, L)] = v1
                plsc.addupdate_scatter(table_v, [v0 * NV + v1], one_f)
                return 0

            lax.fori_loop(0, NF // L, p1_body, 0)

        eps = eps_v[...]
        (x0, y0, z0), (x1, y1, z1) = xyzs

        def p2_body(t, acc):
            v0 = v0buf[pl.ds(t * L, L)]
            v1 = v1buf[pl.ds(t * L, L)]
            cnt = plsc.load_gather(table_v, [v0 * NV + v1])
            w = 1.0 / cnt

            m0 = jnp.logical_and(plsc.load_gather(pres0_v, [v0]) > 0,
                                 plsc.load_gather(pres0_v, [v1]) > 0)
            m1 = jnp.logical_and(plsc.load_gather(pres1_v, [v0]) > 0,
                                 plsc.load_gather(pres1_v, [v1]) > 0)
            val0 = jnp.where(jnp.logical_and(v0 != x0, v1 != x0), x0,
                             jnp.where(jnp.logical_and(v0 != y0, v1 != y0), y0, z0))
            val1 = jnp.where(jnp.logical_and(v0 != x1, v1 != x1), x1,
                             jnp.where(jnp.logical_and(v0 != y1, v1 != y1), y1, z1))
            v2 = jnp.where(m0, val0, jnp.where(m1, val1, 0))
            v3 = jnp.where(jnp.logical_and(m0, m1), val1, 0)

            def vert(vidx):
                b3 = vidx * 3
                return (plsc.load_gather(verts_v, [b3]),
                        plsc.load_gather(verts_v, [b3 + 1]),
                        plsc.load_gather(verts_v, [b3 + 2]))

            p0 = vert(v0)
            p1 = vert(v1)
            p2 = vert(v2)
            p3 = vert(v3)
            a1 = tuple(p1[k] - p0[k] for k in range(3))
            b1 = tuple(p2[k] - p0[k] for k in range(3))
            b2 = tuple(p3[k] - p0[k] for k in range(3))

            def dot3(u, v):
                return u[0] * v[0] + u[1] * v[1] + u[2] * v[2]

            a1l2 = dot3(a1, a1)
            b1l2 = dot3(b1, b1)
            b2l2 = dot3(b2, b2)
            ab1 = dot3(a1, b1)
            ab2 = dot3(a1, b2)
            a1l1 = _sqrt(a1l2 + eps)
            b1l1 = _sqrt(b1l2 + eps)
            b2l1 = _sqrt(b2l2 + eps)
            cos1 = ab1 / (a1l1 * b1l1 + eps)
            sin1 = _sqrt(1.0 - cos1 * cos1 + eps)
            cos2 = ab2 / (a1l1 * b2l1 + eps)
            sin2 = _sqrt(1.0 - cos2 * cos2 + eps)
            r1 = ab1 / (a1l2 + eps)
            r2 = ab2 / (a1l2 + eps)
            cb1 = tuple(b1[k] - a1[k] * r1 for k in range(3))
            cb2 = tuple(b2[k] - a1[k] * r2 for k in range(3))
            cosf = dot3(cb1, cb2) / (b1l1 * sin1 * b2l1 * sin2 + eps)
            term = (cosf + 1.0) * (cosf + 1.0)
            return acc + term * w

        acc = lax.fori_loop(0, NE // L, p2_body, jnp.zeros((L,), jnp.float32))
        acc_v[...] = acc
        pltpu.sync_copy(acc_v, out_hbm.at[cid])


def kernel(vertices, faces, eps):
    verts2d = vertices.reshape(2, 3 * NV).astype(jnp.float32)
    faces2d = faces.reshape(2, FLAT).astype(jnp.int32)
    eps16 = jnp.full((L,), eps, jnp.float32)
    zeros_tbl = jnp.zeros((TBL,), jnp.float32)

    mesh = plsc.VectorSubcoreMesh(core_axis_name="c", subcore_axis_name="s")
    run = pl.kernel(
        _sc_body,
        mesh=mesh,
        out_type=jax.ShapeDtypeStruct((2, L), jnp.float32),
        compiler_params=pltpu.CompilerParams(needs_layout_passes=False),
        scratch_types=[
            pltpu.VMEM((FLAT + L,), jnp.int32),
            pltpu.VMEM((FLAT + L,), jnp.int32),
            pltpu.VMEM((FLAT,), jnp.int32),
            pltpu.VMEM((3 * NV,), jnp.float32),
            pltpu.VMEM((TBL,), jnp.float32),
            pltpu.VMEM((NE,), jnp.int32),
            pltpu.VMEM((NE,), jnp.int32),
            pltpu.VMEM((NV,), jnp.int32),
            pltpu.VMEM((NV,), jnp.int32),
            pltpu.VMEM((L,), jnp.float32),
            pltpu.VMEM((L,), jnp.float32),
        ],
    )
    out = run(verts2d, faces2d, eps16, zeros_tbl)
    return jnp.sum(out)


# speedup vs baseline: 18.6338x; 1.1969x over previous
import functools

import jax
import jax.numpy as jnp
from jax import lax
from jax.experimental import pallas as pl
from jax.experimental.pallas import tpu as pltpu
from jax.experimental.pallas import tpu_sc as plsc

L = 16
NS = 16
NF = 2048
NE = 2 * NF
FLAT = 3 * NF
NV = 256
TBL = NV * NV


def _iota():
    return lax.iota(jnp.int32, L)


def _splat_i(x):
    return jnp.full((L,), x, jnp.int32)


def _rsqrt(v):
    i = plsc.bitcast(v, jnp.int32)
    i = _splat_i(0x5F3759DF) - (i >> 1)
    y = plsc.bitcast(i, jnp.float32)
    for _ in range(3):
        y = y * (1.5 - 0.5 * v * y * y)
    return y


def _sqrt(v):
    return v * _rsqrt(v)


def _splat_at(flat_ref, pos):
    chunk = flat_ref[pl.ds(pos, L)]
    return jnp.full((L,), chunk[0], jnp.int32)


def _first_not_in(flat_ref, xs_v, ys_v):
    def cond(st):
        j, pos = st
        return jnp.logical_and(j < FLAT // L, pos < 0)

    def body(st):
        j, pos = st
        chunk = flat_ref[pl.ds(j * L, L)]
        m = jnp.logical_and(chunk != xs_v, chunk != ys_v)
        c = lax.reduce_min(jnp.where(m, _iota(), _splat_i(L)), axes=(0,))
        pos2 = jnp.where(c < L, j * L + c, -1)
        return j + 1, pos2

    _, pos = lax.while_loop(cond, body, (jnp.int32(0), jnp.int32(-1)))
    pos = jnp.maximum(pos, 0)
    return _splat_at(flat_ref, pos)


def _sc_body(verts_hbm, faces_hbm, eps_hbm, out_hbm,
             faces0_v, faces1_v, myfaces_v, verts_v, table_v,
             v0buf, v1buf, keybuf, pres0_v, pres1_v, eps_v, acc_v):
    cid = lax.axis_index("c")
    sid = lax.axis_index("s")

    pltpu.sync_copy(faces_hbm.at[0], faces0_v.at[pl.ds(0, FLAT)])
    pltpu.sync_copy(faces_hbm.at[1], faces1_v.at[pl.ds(0, FLAT)])
    pltpu.sync_copy(faces_hbm.at[cid], myfaces_v)
    pltpu.sync_copy(verts_hbm.at[cid], verts_v)
    pltpu.sync_copy(eps_hbm, eps_v)

    zero_i = _splat_i(0)
    one_i = _splat_i(1)
    for pres in (pres0_v, pres1_v):
        for j in range(NV // L):
            pres[pl.ds(j * L, L)] = zero_i

    for flat_ref, pres in ((faces0_v, pres0_v), (faces1_v, pres1_v)):
        def pres_body(j, _, flat_ref=flat_ref, pres=pres):
            chunk = flat_ref[pl.ds(j * L, L)]
            plsc.store_scatter(pres, [chunk], one_i)
            return 0

        lax.fori_loop(0, FLAT // L, pres_body, 0)

    xyzs = []
    for flat_ref in (faces0_v, faces1_v):
        xs = _splat_at(flat_ref, 0)
        ys = _first_not_in(flat_ref, xs, xs)
        zs = _first_not_in(flat_ref, xs, ys)
        xyzs.append((xs, ys, zs))

    zero_f = jnp.zeros((L,), jnp.float32)
    one_f = jnp.full((L,), 1.0, jnp.float32)
    for h in range(2):
        def pa_body(t, _, h=h):
            ii = _iota() + t * L
            base = ii * 3 + h
            a = plsc.load_gather(myfaces_v, [base])
            b = plsc.load_gather(myfaces_v, [base + 1])
            v0 = jnp.minimum(a, b)
            v1 = jnp.maximum(a, b)
            key = v0 * NV + v1
            v0buf[pl.ds(h * NF + t * L, L)] = v0
            v1buf[pl.ds(h * NF + t * L, L)] = v1
            keybuf[pl.ds(h * NF + t * L, L)] = key
            plsc.store_scatter(table_v, [key], zero_f)
            return 0

        lax.fori_loop(0, NF // L, pa_body, 0)

    def pb_body(t, _):
        key = keybuf[pl.ds(t * L, L)]
        plsc.addupdate_scatter(table_v, [key], one_f)
        return 0

    lax.fori_loop(0, NE // L, pb_body, 0)

    eps = eps_v[...]
    (x0, y0, z0), (x1, y1, z1) = xyzs

    def pc_body(tt, acc):
        t = sid * (NE // L // NS) + tt
        v0 = v0buf[pl.ds(t * L, L)]
        v1 = v1buf[pl.ds(t * L, L)]
        cnt = plsc.load_gather(table_v, [keybuf[pl.ds(t * L, L)]])
        w = 1.0 / cnt

        m0 = jnp.logical_and(plsc.load_gather(pres0_v, [v0]) > 0,
                             plsc.load_gather(pres0_v, [v1]) > 0)
        m1 = jnp.logical_and(plsc.load_gather(pres1_v, [v0]) > 0,
                             plsc.load_gather(pres1_v, [v1]) > 0)
        val0 = jnp.where(jnp.logical_and(v0 != x0, v1 != x0), x0,
                         jnp.where(jnp.logical_and(v0 != y0, v1 != y0), y0, z0))
        val1 = jnp.where(jnp.logical_and(v0 != x1, v1 != x1), x1,
                         jnp.where(jnp.logical_and(v0 != y1, v1 != y1), y1, z1))
        v2 = jnp.where(m0, val0, jnp.where(m1, val1, 0))
        v3 = jnp.where(jnp.logical_and(m0, m1), val1, 0)

        def vert(vidx):
            b3 = vidx * 3
            return (plsc.load_gather(verts_v, [b3]),
                    plsc.load_gather(verts_v, [b3 + 1]),
                    plsc.load_gather(verts_v, [b3 + 2]))

        p0 = vert(v0)
        p1 = vert(v1)
        p2 = vert(v2)
        p3 = vert(v3)
        a1 = tuple(p1[k] - p0[k] for k in range(3))
        b1 = tuple(p2[k] - p0[k] for k in range(3))
        b2 = tuple(p3[k] - p0[k] for k in range(3))

        def dot3(u, v):
            return u[0] * v[0] + u[1] * v[1] + u[2] * v[2]

        a1l2 = dot3(a1, a1)
        b1l2 = dot3(b1, b1)
        b2l2 = dot3(b2, b2)
        ab1 = dot3(a1, b1)
        ab2 = dot3(a1, b2)
        a1l1 = _sqrt(a1l2 + eps)
        b1l1 = _sqrt(b1l2 + eps)
        b2l1 = _sqrt(b2l2 + eps)
        cos1 = ab1 / (a1l1 * b1l1 + eps)
        sin1 = _sqrt(1.0 - cos1 * cos1 + eps)
        cos2 = ab2 / (a1l1 * b2l1 + eps)
        sin2 = _sqrt(1.0 - cos2 * cos2 + eps)
        r1 = ab1 / (a1l2 + eps)
        r2 = ab2 / (a1l2 + eps)
        cb1 = tuple(b1[k] - a1[k] * r1 for k in range(3))
        cb2 = tuple(b2[k] - a1[k] * r2 for k in range(3))
        cosf = dot3(cb1, cb2) / (b1l1 * sin1 * b2l1 * sin2 + eps)
        term = (cosf + 1.0) * (cosf + 1.0)
        return acc + term * w

    acc = lax.fori_loop(0, NE // L // NS, pc_body, jnp.zeros((L,), jnp.float32))
    acc_v[...] = acc
    pltpu.sync_copy(acc_v, out_hbm.at[cid * NS + sid])


def kernel(vertices, faces, eps):
    verts2d = vertices.reshape(2, 3 * NV).astype(jnp.float32)
    faces2d = faces.reshape(2, FLAT).astype(jnp.int32)
    eps16 = jnp.full((L,), eps, jnp.float32)

    mesh = plsc.VectorSubcoreMesh(core_axis_name="c", subcore_axis_name="s")
    run = pl.kernel(
        _sc_body,
        mesh=mesh,
        out_type=jax.ShapeDtypeStruct((2 * NS, L), jnp.float32),
        compiler_params=pltpu.CompilerParams(needs_layout_passes=False),
        scratch_types=[
            pltpu.VMEM((FLAT + L,), jnp.int32),
            pltpu.VMEM((FLAT + L,), jnp.int32),
            pltpu.VMEM((FLAT,), jnp.int32),
            pltpu.VMEM((3 * NV,), jnp.float32),
            pltpu.VMEM((TBL,), jnp.float32),
            pltpu.VMEM((NE,), jnp.int32),
            pltpu.VMEM((NE,), jnp.int32),
            pltpu.VMEM((NE,), jnp.int32),
            pltpu.VMEM((NV,), jnp.int32),
            pltpu.VMEM((NV,), jnp.int32),
            pltpu.VMEM((L,), jnp.float32),
            pltpu.VMEM((L,), jnp.float32),
        ],
    )
    out = run(verts2d, faces2d, eps16)
    return jnp.sum(out)


# speedup vs baseline: 23.0447x; 1.2367x over previous
import functools

import jax
import jax.numpy as jnp
from jax import lax
from jax.experimental import pallas as pl
from jax.experimental.pallas import tpu as pltpu
from jax.experimental.pallas import tpu_sc as plsc

L = 16
NS = 16
NF = 2048
NE = 2 * NF
FLAT = 3 * NF
NV = 256
TBL = NV * NV
FS = NF // NS
WS = 3 * FS
ES = 2 * FS


def _iota():
    return lax.iota(jnp.int32, L)


def _splat_i(x):
    return jnp.full((L,), x, jnp.int32)


def _rsqrt(v):
    i = plsc.bitcast(v, jnp.int32)
    i = _splat_i(0x5F3759DF) - (i >> 1)
    y = plsc.bitcast(i, jnp.float32)
    for _ in range(3):
        y = y * (1.5 - 0.5 * v * y * y)
    return y


def _sqrt(v):
    return v * _rsqrt(v)


def _splat_at(flat_ref, pos):
    chunk = flat_ref[pl.ds(pos, L)]
    return jnp.full((L,), chunk[0], jnp.int32)


def _first_not_in(flat_ref, xs_v, ys_v):
    def cond(st):
        j, pos = st
        return jnp.logical_and(j < FLAT // L, pos < 0)

    def body(st):
        j, pos = st
        chunk = flat_ref[pl.ds(j * L, L)]
        m = jnp.logical_and(chunk != xs_v, chunk != ys_v)
        c = lax.reduce_min(jnp.where(m, _iota(), _splat_i(L)), axes=(0,))
        pos2 = jnp.where(c < L, j * L + c, -1)
        return j + 1, pos2

    _, pos = lax.while_loop(cond, body, (jnp.int32(0), jnp.int32(-1)))
    pos = jnp.maximum(pos, 0)
    return _splat_at(flat_ref, pos)


def _sc_body(verts_hbm, faces_hbm, eps_hbm, out_hbm,
             sh_table, sh_pres, sh_xyz,
             f0slice, presidx1, myslice, verts_v,
             keyidx, v0s, v1s, cntbuf, presfull, xyzbuf,
             flat0_v, flat1_v, zeros_f, ones_f, ones_i, zeros_i, eps_v, acc_v):
    cid = lax.axis_index("c")
    sid = lax.axis_index("s")

    base0 = sid * WS
    for k in range(3):
        pltpu.sync_copy(faces_hbm.at[pl.ds(0 * FLAT + base0 + k * FS, FS)],
                        f0slice.at[k])
        pltpu.sync_copy(faces_hbm.at[pl.ds(1 * FLAT + base0 + k * FS, FS)],
                        presidx1.at[k])
    pltpu.sync_copy(faces_hbm.at[pl.ds(cid * FLAT + base0, WS)], myslice)
    pltpu.sync_copy(verts_hbm.at[cid], verts_v)
    pltpu.sync_copy(eps_hbm, eps_v)

    zf = jnp.zeros((L,), jnp.float32)
    zi = _splat_i(0)
    oi = _splat_i(1)
    of = jnp.full((L,), 1.0, jnp.float32)
    for j in range(FS // L):
        zeros_f[pl.ds(j * L, L)] = zf
        ones_f[pl.ds(j * L, L)] = of
        ones_i[pl.ds(j * L, L)] = oi
    for j in range(2 * NV // NS // L):
        zeros_i[pl.ds(j * L, L)] = zi

    pltpu.sync_copy(zeros_i, sh_pres.at[pl.ds(sid * (2 * NV // NS), 2 * NV // NS)])

    for k in range(3):
        for j in range(FS // L):
            presidx1[k, pl.ds(j * L, L)] = presidx1[k, pl.ds(j * L, L)] + NV

    for h in range(2):
        def pa_body(t, _, h=h):
            jj = _iota() + t * L
            base = jj * 3 + h
            a = plsc.load_gather(myslice, [base])
            b = plsc.load_gather(myslice, [base + 1])
            v0 = jnp.minimum(a, b)
            v1 = jnp.maximum(a, b)
            v0s[pl.ds(h * FS + t * L, L)] = v0
            v1s[pl.ds(h * FS + t * L, L)] = v1
            keyidx[h, pl.ds(t * L, L)] = v0 * NV + v1
            return 0

        lax.fori_loop(0, FS // L, pa_body, 0)

    for j in range(2):
        pltpu.sync_copy(zeros_f, sh_table.at[keyidx.at[j]])

    @pl.when(sid == 0)
    def _():
        pltpu.sync_copy(faces_hbm.at[pl.ds(0, FLAT)], flat0_v.at[pl.ds(0, FLAT)])
        pltpu.sync_copy(faces_hbm.at[pl.ds(FLAT, FLAT)], flat1_v.at[pl.ds(0, FLAT)])
        vec = _splat_i(0)
        lanes = _iota()
        for slot, flat_ref in enumerate((flat0_v, flat1_v)):
            xs = _splat_at(flat_ref, 0)
            ys = _first_not_in(flat_ref, xs, xs)
            zs = _first_not_in(flat_ref, xs, ys)
            vec = jnp.where(lanes == 3 * slot, xs, vec)
            vec = jnp.where(lanes == 3 * slot + 1, ys, vec)
            vec = jnp.where(lanes == 3 * slot + 2, zs, vec)
        xyzbuf[...] = vec
        pltpu.sync_copy(xyzbuf, sh_xyz)

    plsc.subcore_barrier()

    for k in range(3):
        pltpu.sync_copy(ones_i, sh_pres.at[f0slice.at[k]])
        pltpu.sync_copy(ones_i, sh_pres.at[presidx1.at[k]])
    for j in range(2):
        pltpu.sync_copy(ones_f, sh_table.at[keyidx.at[j]], add=True)

    plsc.subcore_barrier()

    pltpu.sync_copy(sh_pres, presfull)
    for j in range(2):
        pltpu.sync_copy(sh_table.at[keyidx.at[j]], cntbuf.at[pl.ds(j * FS, FS)])
    pltpu.sync_copy(sh_xyz, xyzbuf)

    xv = xyzbuf[...]
    x0 = jnp.full((L,), xv[0], jnp.int32)
    y0 = jnp.full((L,), xv[1], jnp.int32)
    z0 = jnp.full((L,), xv[2], jnp.int32)
    x1 = jnp.full((L,), xv[3], jnp.int32)
    y1 = jnp.full((L,), xv[4], jnp.int32)
    z1 = jnp.full((L,), xv[5], jnp.int32)
    eps = eps_v[...]

    def pc_body(tt, acc):
        v0 = v0s[pl.ds(tt * L, L)]
        v1 = v1s[pl.ds(tt * L, L)]
        cnt = cntbuf[pl.ds(tt * L, L)]
        w = 1.0 / cnt

        m0 = jnp.logical_and(plsc.load_gather(presfull, [v0]) > 0,
                             plsc.load_gather(presfull, [v1]) > 0)
        m1 = jnp.logical_and(plsc.load_gather(presfull, [v0 + NV]) > 0,
                             plsc.load_gather(presfull, [v1 + NV]) > 0)
        val0 = jnp.where(jnp.logical_and(v0 != x0, v1 != x0), x0,
                         jnp.where(jnp.logical_and(v0 != y0, v1 != y0), y0, z0))
        val1 = jnp.where(jnp.logical_and(v0 != x1, v1 != x1), x1,
                         jnp.where(jnp.logical_and(v0 != y1, v1 != y1), y1, z1))
        v2 = jnp.where(m0, val0, jnp.where(m1, val1, 0))
        v3 = jnp.where(jnp.logical_and(m0, m1), val1, 0)

        def vert(vidx):
            b3 = vidx * 3
            return (plsc.load_gather(verts_v, [b3]),
                    plsc.load_gather(verts_v, [b3 + 1]),
                    plsc.load_gather(verts_v, [b3 + 2]))

        p0 = vert(v0)
        p1 = vert(v1)
        p2 = vert(v2)
        p3 = vert(v3)
        a1 = tuple(p1[k] - p0[k] for k in range(3))
        b1 = tuple(p2[k] - p0[k] for k in range(3))
        b2 = tuple(p3[k] - p0[k] for k in range(3))

        def dot3(u, v):
            return u[0] * v[0] + u[1] * v[1] + u[2] * v[2]

        a1l2 = dot3(a1, a1)
        b1l2 = dot3(b1, b1)
        b2l2 = dot3(b2, b2)
        ab1 = dot3(a1, b1)
        ab2 = dot3(a1, b2)
        a1l1 = _sqrt(a1l2 + eps)
        b1l1 = _sqrt(b1l2 + eps)
        b2l1 = _sqrt(b2l2 + eps)
        cos1 = ab1 / (a1l1 * b1l1 + eps)
        sin1 = _sqrt(1.0 - cos1 * cos1 + eps)
        cos2 = ab2 / (a1l1 * b2l1 + eps)
        sin2 = _sqrt(1.0 - cos2 * cos2 + eps)
        r1 = ab1 / (a1l2 + eps)
        r2 = ab2 / (a1l2 + eps)
        cb1 = tuple(b1[k] - a1[k] * r1 for k in range(3))
        cb2 = tuple(b2[k] - a1[k] * r2 for k in range(3))
        cosf = dot3(cb1, cb2) / (b1l1 * sin1 * b2l1 * sin2 + eps)
        term = (cosf + 1.0) * (cosf + 1.0)
        return acc + term * w

    acc = lax.fori_loop(0, ES // L, pc_body, jnp.zeros((L,), jnp.float32))
    acc_v[...] = acc
    pltpu.sync_copy(acc_v, out_hbm.at[cid * NS + sid])


def kernel(vertices, faces, eps):
    verts2d = vertices.reshape(2, 3 * NV).astype(jnp.float32)
    faces1d = faces.reshape(2 * FLAT).astype(jnp.int32)
    eps16 = jnp.full((L,), eps, jnp.float32)

    mesh = plsc.VectorSubcoreMesh(core_axis_name="c", subcore_axis_name="s")
    run = pl.kernel(
        _sc_body,
        mesh=mesh,
        out_type=jax.ShapeDtypeStruct((2 * NS, L), jnp.float32),
        compiler_params=pltpu.CompilerParams(needs_layout_passes=False),
        scratch_types=[
            pltpu.VMEM_SHARED((TBL,), jnp.float32),
            pltpu.VMEM_SHARED((2 * NV,), jnp.int32),
            pltpu.VMEM_SHARED((L,), jnp.int32),
            pltpu.VMEM((3, FS), jnp.int32),
            pltpu.VMEM((3, FS), jnp.int32),
            pltpu.VMEM((WS,), jnp.int32),
            pltpu.VMEM((3 * NV,), jnp.float32),
            pltpu.VMEM((2, FS), jnp.int32),
            pltpu.VMEM((ES,), jnp.int32),
            pltpu.VMEM((ES,), jnp.int32),
            pltpu.VMEM((ES,), jnp.float32),
            pltpu.VMEM((2 * NV,), jnp.int32),
            pltpu.VMEM((L,), jnp.int32),
            pltpu.VMEM((FLAT + L,), jnp.int32),
            pltpu.VMEM((FLAT + L,), jnp.int32),
            pltpu.VMEM((FS,), jnp.float32),
            pltpu.VMEM((FS,), jnp.float32),
            pltpu.VMEM((FS,), jnp.int32),
            pltpu.VMEM((2 * NV // NS,), jnp.int32),
            pltpu.VMEM((L,), jnp.float32),
            pltpu.VMEM((L,), jnp.float32),
        ],
    )
    out = run(verts2d, faces1d, eps16)
    return jnp.sum(out)


# speedup vs baseline: 26.8473x; 1.1650x over previous
import functools

import jax
import jax.numpy as jnp
from jax import lax
from jax.experimental import pallas as pl
from jax.experimental.pallas import tpu as pltpu
from jax.experimental.pallas import tpu_sc as plsc

L = 16
NS = 16
NF = 2048
NE = 2 * NF
FLAT = 3 * NF
NV = 256
TBL = NV * NV
FS = NF // NS
WS = 3 * FS
ES = 2 * FS


def _iota():
    return lax.iota(jnp.int32, L)


def _splat_i(x):
    return jnp.full((L,), x, jnp.int32)


def _rsqrt(v):
    i = plsc.bitcast(v, jnp.int32)
    i = _splat_i(0x5F3759DF) - (i >> 1)
    y = plsc.bitcast(i, jnp.float32)
    for _ in range(3):
        y = y * (1.5 - 0.5 * v * y * y)
    return y


def _sqrt(v):
    return v * _rsqrt(v)


def _splat_at(flat_ref, pos):
    chunk = flat_ref[pl.ds(pos, L)]
    return jnp.full((L,), chunk[0], jnp.int32)


def _first_not_in(flat_ref, xs_v, ys_v):
    def cond(st):
        j, pos = st
        return jnp.logical_and(j < FLAT // L, pos < 0)

    def body(st):
        j, pos = st
        chunk = flat_ref[pl.ds(j * L, L)]
        m = jnp.logical_and(chunk != xs_v, chunk != ys_v)
        c = lax.reduce_min(jnp.where(m, _iota(), _splat_i(L)), axes=(0,))
        pos2 = jnp.where(c < L, j * L + c, -1)
        return j + 1, pos2

    _, pos = lax.while_loop(cond, body, (jnp.int32(0), jnp.int32(-1)))
    pos = jnp.maximum(pos, 0)
    return _splat_at(flat_ref, pos)


def _sc_body(verts_hbm, faces_hbm, eps_hbm, out_hbm,
             sh_table, sh_pres, sh_xyz,
             f0slice, presidx1, myslice, verts_v,
             keyidx, v0s, v1s, cntbuf, presfull, xyzbuf,
             flat0_v, flat1_v, zeros_f, ones_f, ones_i, zeros_i, eps_v, acc_v,
             sem, sem1, sem2):
    cid = lax.axis_index("c")
    sid = lax.axis_index("s")

    base0 = sid * WS
    stage = []
    for k in range(3):
        stage.append(pltpu.async_copy(
            faces_hbm.at[pl.ds(0 * FLAT + base0 + k * FS, FS)], f0slice.at[k], sem))
        stage.append(pltpu.async_copy(
            faces_hbm.at[pl.ds(1 * FLAT + base0 + k * FS, FS)], presidx1.at[k], sem))
    stage.append(pltpu.async_copy(
        faces_hbm.at[pl.ds(cid * FLAT + base0, WS)], myslice, sem))
    stage.append(pltpu.async_copy(verts_hbm.at[cid], verts_v, sem))
    stage.append(pltpu.async_copy(eps_hbm, eps_v, sem))

    zf = jnp.zeros((L,), jnp.float32)
    zi = _splat_i(0)
    oi = _splat_i(1)
    of = jnp.full((L,), 1.0, jnp.float32)
    for j in range(FS // L):
        zeros_f[pl.ds(j * L, L)] = zf
        ones_f[pl.ds(j * L, L)] = of
        ones_i[pl.ds(j * L, L)] = oi
    for j in range(2 * NV // NS // L):
        zeros_i[pl.ds(j * L, L)] = zi

    zp = pltpu.async_copy(
        zeros_i, sh_pres.at[pl.ds(sid * (2 * NV // NS), 2 * NV // NS)], sem1)
    for cp in stage:
        cp.wait()

    for k in range(3):
        for j in range(FS // L):
            presidx1[k, pl.ds(j * L, L)] = presidx1[k, pl.ds(j * L, L)] + NV

    for h in range(2):
        def pa_body(t, _, h=h):
            jj = _iota() + t * L
            base = jj * 3 + h
            a = plsc.load_gather(myslice, [base])
            b = plsc.load_gather(myslice, [base + 1])
            v0 = jnp.minimum(a, b)
            v1 = jnp.maximum(a, b)
            v0s[pl.ds(h * FS + t * L, L)] = v0
            v1s[pl.ds(h * FS + t * L, L)] = v1
            keyidx[h, pl.ds(t * L, L)] = v0 * NV + v1
            return 0

        lax.fori_loop(0, FS // L, pa_body, 0)

    zk = [pltpu.async_copy(zeros_f, sh_table.at[keyidx.at[j]], sem1)
          for j in range(2)]

    @pl.when(sid == 0)
    def _():
        c0 = pltpu.async_copy(faces_hbm.at[pl.ds(0, FLAT)],
                              flat0_v.at[pl.ds(0, FLAT)], sem2)
        c1 = pltpu.async_copy(faces_hbm.at[pl.ds(FLAT, FLAT)],
                              flat1_v.at[pl.ds(0, FLAT)], sem2)
        c0.wait()
        c1.wait()
        vec = _splat_i(0)
        lanes = _iota()
        for slot, flat_ref in enumerate((flat0_v, flat1_v)):
            xs = _splat_at(flat_ref, 0)
            ys = _first_not_in(flat_ref, xs, xs)
            zs = _first_not_in(flat_ref, xs, ys)
            vec = jnp.where(lanes == 3 * slot, xs, vec)
            vec = jnp.where(lanes == 3 * slot + 1, ys, vec)
            vec = jnp.where(lanes == 3 * slot + 2, zs, vec)
        xyzbuf[...] = vec
        pltpu.sync_copy(xyzbuf, sh_xyz)

    zp.wait()
    for cp in zk:
        cp.wait()
    plsc.subcore_barrier()

    acc_cps = []
    for k in range(3):
        acc_cps.append(pltpu.async_copy(ones_i, sh_pres.at[f0slice.at[k]], sem))
        acc_cps.append(pltpu.async_copy(ones_i, sh_pres.at[presidx1.at[k]], sem))
    for j in range(2):
        acc_cps.append(pltpu.async_copy(ones_f, sh_table.at[keyidx.at[j]],
                                        sem, add=True))
    for cp in acc_cps:
        cp.wait()

    plsc.subcore_barrier()

    rd = [pltpu.async_copy(sh_pres, presfull, sem),
          pltpu.async_copy(sh_xyz, xyzbuf, sem)]
    rd += [pltpu.async_copy(sh_table.at[keyidx.at[j]],
                            cntbuf.at[pl.ds(j * FS, FS)], sem)
           for j in range(2)]
    for cp in rd:
        cp.wait()

    xv = xyzbuf[...]
    x0 = jnp.full((L,), xv[0], jnp.int32)
    y0 = jnp.full((L,), xv[1], jnp.int32)
    z0 = jnp.full((L,), xv[2], jnp.int32)
    x1 = jnp.full((L,), xv[3], jnp.int32)
    y1 = jnp.full((L,), xv[4], jnp.int32)
    z1 = jnp.full((L,), xv[5], jnp.int32)
    eps = eps_v[...]

    def pc_body(tt, acc):
        v0 = v0s[pl.ds(tt * L, L)]
        v1 = v1s[pl.ds(tt * L, L)]
        cnt = cntbuf[pl.ds(tt * L, L)]
        w = 1.0 / cnt

        m0 = jnp.logical_and(plsc.load_gather(presfull, [v0]) > 0,
                             plsc.load_gather(presfull, [v1]) > 0)
        m1 = jnp.logical_and(plsc.load_gather(presfull, [v0 + NV]) > 0,
                             plsc.load_gather(presfull, [v1 + NV]) > 0)
        val0 = jnp.where(jnp.logical_and(v0 != x0, v1 != x0), x0,
                         jnp.where(jnp.logical_and(v0 != y0, v1 != y0), y0, z0))
        val1 = jnp.where(jnp.logical_and(v0 != x1, v1 != x1), x1,
                         jnp.where(jnp.logical_and(v0 != y1, v1 != y1), y1, z1))
        v2 = jnp.where(m0, val0, jnp.where(m1, val1, 0))
        v3 = jnp.where(jnp.logical_and(m0, m1), val1, 0)

        def vert(vidx):
            b3 = vidx * 3
            return (plsc.load_gather(verts_v, [b3]),
                    plsc.load_gather(verts_v, [b3 + 1]),
                    plsc.load_gather(verts_v, [b3 + 2]))

        p0 = vert(v0)
        p1 = vert(v1)
        p2 = vert(v2)
        p3 = vert(v3)
        a1 = tuple(p1[k] - p0[k] for k in range(3))
        b1 = tuple(p2[k] - p0[k] for k in range(3))
        b2 = tuple(p3[k] - p0[k] for k in range(3))

        def dot3(u, v):
            return u[0] * v[0] + u[1] * v[1] + u[2] * v[2]

        a1l2 = dot3(a1, a1)
        b1l2 = dot3(b1, b1)
        b2l2 = dot3(b2, b2)
        ab1 = dot3(a1, b1)
        ab2 = dot3(a1, b2)
        a1l1 = _sqrt(a1l2 + eps)
        b1l1 = _sqrt(b1l2 + eps)
        b2l1 = _sqrt(b2l2 + eps)
        cos1 = ab1 / (a1l1 * b1l1 + eps)
        sin1 = _sqrt(1.0 - cos1 * cos1 + eps)
        cos2 = ab2 / (a1l1 * b2l1 + eps)
        sin2 = _sqrt(1.0 - cos2 * cos2 + eps)
        r1 = ab1 / (a1l2 + eps)
        r2 = ab2 / (a1l2 + eps)
        cb1 = tuple(b1[k] - a1[k] * r1 for k in range(3))
        cb2 = tuple(b2[k] - a1[k] * r2 for k in range(3))
        cosf = dot3(cb1, cb2) / (b1l1 * sin1 * b2l1 * sin2 + eps)
        term = (cosf + 1.0) * (cosf + 1.0)
        return acc + term * w

    acc = lax.fori_loop(0, ES // L, pc_body, jnp.zeros((L,), jnp.float32))
    acc_v[...] = acc
    pltpu.sync_copy(acc_v, out_hbm.at[cid * NS + sid])


def kernel(vertices, faces, eps):
    verts2d = vertices.reshape(2, 3 * NV).astype(jnp.float32)
    faces1d = faces.reshape(2 * FLAT).astype(jnp.int32)
    eps16 = jnp.full((L,), eps, jnp.float32)

    mesh = plsc.VectorSubcoreMesh(core_axis_name="c", subcore_axis_name="s")
    run = pl.kernel(
        _sc_body,
        mesh=mesh,
        out_type=jax.ShapeDtypeStruct((2 * NS, L), jnp.float32),
        compiler_params=pltpu.CompilerParams(needs_layout_passes=False),
        scratch_types=[
            pltpu.VMEM_SHARED((TBL,), jnp.float32),
            pltpu.VMEM_SHARED((2 * NV,), jnp.int32),
            pltpu.VMEM_SHARED((L,), jnp.int32),
            pltpu.VMEM((3, FS), jnp.int32),
            pltpu.VMEM((3, FS), jnp.int32),
            pltpu.VMEM((WS,), jnp.int32),
            pltpu.VMEM((3 * NV,), jnp.float32),
            pltpu.VMEM((2, FS), jnp.int32),
            pltpu.VMEM((ES,), jnp.int32),
            pltpu.VMEM((ES,), jnp.int32),
            pltpu.VMEM((ES,), jnp.float32),
            pltpu.VMEM((2 * NV,), jnp.int32),
            pltpu.VMEM((L,), jnp.int32),
            pltpu.VMEM((FLAT + L,), jnp.int32),
            pltpu.VMEM((FLAT + L,), jnp.int32),
            pltpu.VMEM((FS,), jnp.float32),
            pltpu.VMEM((FS,), jnp.float32),
            pltpu.VMEM((FS,), jnp.int32),
            pltpu.VMEM((2 * NV // NS,), jnp.int32),
            pltpu.VMEM((L,), jnp.float32),
            pltpu.VMEM((L,), jnp.float32),
            pltpu.SemaphoreType.DMA,
            pltpu.SemaphoreType.DMA,
            pltpu.SemaphoreType.DMA,
        ],
    )
    out = run(verts2d, faces1d, eps16)
    return jnp.sum(out)


# speedup vs baseline: 26.8988x; 1.0019x over previous
import functools

import jax
import jax.numpy as jnp
from jax import lax
from jax.experimental import pallas as pl
from jax.experimental.pallas import tpu as pltpu
from jax.experimental.pallas import tpu_sc as plsc

L = 16
NS = 16
NF = 2048
NE = 2 * NF
FLAT = 3 * NF
NV = 256
TBL = NV * NV
FS = NF // NS
WS = 3 * FS
ES = 2 * FS


def _iota():
    return lax.iota(jnp.int32, L)


def _splat_i(x):
    return jnp.full((L,), x, jnp.int32)


def _rsqrt(v):
    i = plsc.bitcast(v, jnp.int32)
    i = _splat_i(0x5F3759DF) - (i >> 1)
    y = plsc.bitcast(i, jnp.float32)
    for _ in range(3):
        y = y * (1.5 - 0.5 * v * y * y)
    return y


def _sqrt(v):
    return v * _rsqrt(v)


def _splat_at(flat_ref, pos):
    chunk = flat_ref[pl.ds(pos, L)]
    return jnp.full((L,), chunk[0], jnp.int32)


def _first_not_in(flat_ref, xs_v, ys_v):
    def cond(st):
        j, pos = st
        return jnp.logical_and(j < FLAT // L, pos < 0)

    def body(st):
        j, pos = st
        chunk = flat_ref[pl.ds(j * L, L)]
        m = jnp.logical_and(chunk != xs_v, chunk != ys_v)
        c = lax.reduce_min(jnp.where(m, _iota(), _splat_i(L)), axes=(0,))
        pos2 = jnp.where(c < L, j * L + c, -1)
        return j + 1, pos2

    _, pos = lax.while_loop(cond, body, (jnp.int32(0), jnp.int32(-1)))
    pos = jnp.maximum(pos, 0)
    return _splat_at(flat_ref, pos)


def _sc_body(verts_hbm, faces_hbm, eps_hbm, out_hbm,
             sh_table, sh_pres, sh_xyz,
             f0slice, presidx1, myslice, verts_v,
             keyidx, v0s, v1s, cntbuf, presfull, xyzbuf,
             flat0_v, flat1_v, zeros_f, ones_f, ones_i, zeros_i, eps_v, acc_v,
             sem, sem1, sem2):
    cid = lax.axis_index("c")
    sid = lax.axis_index("s")

    base0 = sid * WS
    stage = []
    for k in range(3):
        stage.append(pltpu.async_copy(
            faces_hbm.at[pl.ds(0 * FLAT + base0 + k * FS, FS)], f0slice.at[k], sem))
        stage.append(pltpu.async_copy(
            faces_hbm.at[pl.ds(1 * FLAT + base0 + k * FS, FS)], presidx1.at[k], sem))
    stage.append(pltpu.async_copy(
        faces_hbm.at[pl.ds(cid * FLAT + base0, WS)], myslice, sem))
    stage.append(pltpu.async_copy(verts_hbm.at[cid], verts_v, sem))
    stage.append(pltpu.async_copy(eps_hbm, eps_v, sem))

    zf = jnp.zeros((L,), jnp.float32)
    zi = _splat_i(0)
    oi = _splat_i(1)
    of = jnp.full((L,), 1.0, jnp.float32)

    def fill_body(j, _):
        zeros_f[pl.ds(j * L, L)] = zf
        ones_f[pl.ds(j * L, L)] = of
        ones_i[pl.ds(j * L, L)] = oi
        return 0

    lax.fori_loop(0, FS // L, fill_body, 0)
    for j in range(2 * NV // NS // L):
        zeros_i[pl.ds(j * L, L)] = zi

    zp = pltpu.async_copy(
        zeros_i, sh_pres.at[pl.ds(sid * (2 * NV // NS), 2 * NV // NS)], sem1)
    for cp in stage:
        cp.wait()

    for k in range(3):
        def shift_body(j, _, k=k):
            presidx1[k, pl.ds(j * L, L)] = presidx1[k, pl.ds(j * L, L)] + NV
            return 0

        lax.fori_loop(0, FS // L, shift_body, 0)

    for h in range(2):
        def pa_body(t, _, h=h):
            jj = _iota() + t * L
            base = jj * 3 + h
            a = plsc.load_gather(myslice, [base])
            b = plsc.load_gather(myslice, [base + 1])
            v0 = jnp.minimum(a, b)
            v1 = jnp.maximum(a, b)
            v0s[pl.ds(h * FS + t * L, L)] = v0
            v1s[pl.ds(h * FS + t * L, L)] = v1
            keyidx[h, pl.ds(t * L, L)] = v0 * NV + v1
            return 0

        lax.fori_loop(0, FS // L, pa_body, 0)

    zk = [pltpu.async_copy(zeros_f, sh_table.at[keyidx.at[j]], sem1)
          for j in range(2)]

    @pl.when(sid == 0)
    def _():
        c0 = pltpu.async_copy(faces_hbm.at[pl.ds(0, FLAT)],
                              flat0_v.at[pl.ds(0, FLAT)], sem2)
        c1 = pltpu.async_copy(faces_hbm.at[pl.ds(FLAT, FLAT)],
                              flat1_v.at[pl.ds(0, FLAT)], sem2)
        c0.wait()
        c1.wait()
        vec = _splat_i(0)
        lanes = _iota()
        for slot, flat_ref in enumerate((flat0_v, flat1_v)):
            xs = _splat_at(flat_ref, 0)
            ys = _first_not_in(flat_ref, xs, xs)
            zs = _first_not_in(flat_ref, xs, ys)
            vec = jnp.where(lanes == 3 * slot, xs, vec)
            vec = jnp.where(lanes == 3 * slot + 1, ys, vec)
            vec = jnp.where(lanes == 3 * slot + 2, zs, vec)
        xyzbuf[...] = vec
        pltpu.sync_copy(xyzbuf, sh_xyz)

    zp.wait()
    for cp in zk:
        cp.wait()
    plsc.subcore_barrier()

    acc_cps = []
    for k in range(3):
        acc_cps.append(pltpu.async_copy(ones_i, sh_pres.at[f0slice.at[k]], sem))
        acc_cps.append(pltpu.async_copy(ones_i, sh_pres.at[presidx1.at[k]], sem))
    for j in range(2):
        acc_cps.append(pltpu.async_copy(ones_f, sh_table.at[keyidx.at[j]],
                                        sem, add=True))
    for cp in acc_cps:
        cp.wait()

    plsc.subcore_barrier()

    rd = [pltpu.async_copy(sh_pres, presfull, sem),
          pltpu.async_copy(sh_xyz, xyzbuf, sem)]
    rd += [pltpu.async_copy(sh_table.at[keyidx.at[j]],
                            cntbuf.at[pl.ds(j * FS, FS)], sem)
           for j in range(2)]
    for cp in rd:
        cp.wait()

    xv = xyzbuf[...]
    x0 = jnp.full((L,), xv[0], jnp.int32)
    y0 = jnp.full((L,), xv[1], jnp.int32)
    z0 = jnp.full((L,), xv[2], jnp.int32)
    x1 = jnp.full((L,), xv[3], jnp.int32)
    y1 = jnp.full((L,), xv[4], jnp.int32)
    z1 = jnp.full((L,), xv[5], jnp.int32)
    eps = eps_v[...]

    def pc_body(tt, acc):
        v0 = v0s[pl.ds(tt * L, L)]
        v1 = v1s[pl.ds(tt * L, L)]
        cnt = cntbuf[pl.ds(tt * L, L)]
        w = 1.0 / cnt

        m0 = jnp.logical_and(plsc.load_gather(presfull, [v0]) > 0,
                             plsc.load_gather(presfull, [v1]) > 0)
        m1 = jnp.logical_and(plsc.load_gather(presfull, [v0 + NV]) > 0,
                             plsc.load_gather(presfull, [v1 + NV]) > 0)
        val0 = jnp.where(jnp.logical_and(v0 != x0, v1 != x0), x0,
                         jnp.where(jnp.logical_and(v0 != y0, v1 != y0), y0, z0))
        val1 = jnp.where(jnp.logical_and(v0 != x1, v1 != x1), x1,
                         jnp.where(jnp.logical_and(v0 != y1, v1 != y1), y1, z1))
        v2 = jnp.where(m0, val0, jnp.where(m1, val1, 0))
        v3 = jnp.where(jnp.logical_and(m0, m1), val1, 0)

        def vert(vidx):
            b3 = vidx * 3
            return (plsc.load_gather(verts_v, [b3]),
                    plsc.load_gather(verts_v, [b3 + 1]),
                    plsc.load_gather(verts_v, [b3 + 2]))

        p0 = vert(v0)
        p1 = vert(v1)
        p2 = vert(v2)
        p3 = vert(v3)
        a1 = tuple(p1[k] - p0[k] for k in range(3))
        b1 = tuple(p2[k] - p0[k] for k in range(3))
        b2 = tuple(p3[k] - p0[k] for k in range(3))

        def dot3(u, v):
            return u[0] * v[0] + u[1] * v[1] + u[2] * v[2]

        a1l2 = dot3(a1, a1)
        b1l2 = dot3(b1, b1)
        b2l2 = dot3(b2, b2)
        ab1 = dot3(a1, b1)
        ab2 = dot3(a1, b2)
        a1l1 = _sqrt(a1l2 + eps)
        b1l1 = _sqrt(b1l2 + eps)
        b2l1 = _sqrt(b2l2 + eps)
        cos1 = ab1 / (a1l1 * b1l1 + eps)
        sin1 = _sqrt(1.0 - cos1 * cos1 + eps)
        cos2 = ab2 / (a1l1 * b2l1 + eps)
        sin2 = _sqrt(1.0 - cos2 * cos2 + eps)
        r1 = ab1 / (a1l2 + eps)
        r2 = ab2 / (a1l2 + eps)
        cb1 = tuple(b1[k] - a1[k] * r1 for k in range(3))
        cb2 = tuple(b2[k] - a1[k] * r2 for k in range(3))
        cosf = dot3(cb1, cb2) / (b1l1 * sin1 * b2l1 * sin2 + eps)
        term = (cosf + 1.0) * (cosf + 1.0)
        return acc + term * w

    acc = lax.fori_loop(0, ES // L, pc_body, jnp.zeros((L,), jnp.float32))
    acc_v[...] = acc
    pltpu.sync_copy(acc_v, out_hbm.at[cid * NS + sid])


def kernel(vertices, faces, eps):
    verts2d = vertices.reshape(2, 3 * NV).astype(jnp.float32)
    faces1d = faces.reshape(2 * FLAT).astype(jnp.int32)
    eps16 = jnp.full((L,), eps, jnp.float32)

    mesh = plsc.VectorSubcoreMesh(core_axis_name="c", subcore_axis_name="s")
    run = pl.kernel(
        _sc_body,
        mesh=mesh,
        out_type=jax.ShapeDtypeStruct((2 * NS, L), jnp.float32),
        compiler_params=pltpu.CompilerParams(needs_layout_passes=False),
        scratch_types=[
            pltpu.VMEM_SHARED((TBL,), jnp.float32),
            pltpu.VMEM_SHARED((2 * NV,), jnp.int32),
            pltpu.VMEM_SHARED((L,), jnp.int32),
            pltpu.VMEM((3, FS), jnp.int32),
            pltpu.VMEM((3, FS), jnp.int32),
            pltpu.VMEM((WS,), jnp.int32),
            pltpu.VMEM((3 * NV,), jnp.float32),
            pltpu.VMEM((2, FS), jnp.int32),
            pltpu.VMEM((ES,), jnp.int32),
            pltpu.VMEM((ES,), jnp.int32),
            pltpu.VMEM((ES,), jnp.float32),
            pltpu.VMEM((2 * NV,), jnp.int32),
            pltpu.VMEM((L,), jnp.int32),
            pltpu.VMEM((FLAT + L,), jnp.int32),
            pltpu.VMEM((FLAT + L,), jnp.int32),
            pltpu.VMEM((FS,), jnp.float32),
            pltpu.VMEM((FS,), jnp.float32),
            pltpu.VMEM((FS,), jnp.int32),
            pltpu.VMEM((2 * NV // NS,), jnp.int32),
            pltpu.VMEM((L,), jnp.float32),
            pltpu.VMEM((L,), jnp.float32),
            pltpu.SemaphoreType.DMA,
            pltpu.SemaphoreType.DMA,
            pltpu.SemaphoreType.DMA,
        ],
    )
    out = run(verts2d, faces1d, eps16)
    return jnp.sum(out)


# speedup vs baseline: 27.8750x; 1.0363x over previous
import functools

import jax
import jax.numpy as jnp
from jax import lax
from jax.experimental import pallas as pl
from jax.experimental.pallas import tpu as pltpu
from jax.experimental.pallas import tpu_sc as plsc

L = 16
NS = 16
NF = 2048
NE = 2 * NF
FLAT = 3 * NF
NV = 256
TBL = NV * NV
FS = NF // NS
WS = 3 * FS
ES = 2 * FS


def _iota():
    return lax.iota(jnp.int32, L)


def _splat_i(x):
    return jnp.full((L,), x, jnp.int32)


def _rsqrt(v):
    i = plsc.bitcast(v, jnp.int32)
    i = _splat_i(0x5F3759DF) - (i >> 1)
    y = plsc.bitcast(i, jnp.float32)
    for _ in range(3):
        y = y * (1.5 - 0.5 * v * y * y)
    return y


def _sqrt(v):
    return v * _rsqrt(v)


def _splat_at(flat_ref, pos):
    chunk = flat_ref[pl.ds(pos, L)]
    return jnp.full((L,), chunk[0], jnp.int32)


def _first_not_in(flat_ref, xs_v, ys_v):
    def cond(st):
        j, pos = st
        return jnp.logical_and(j < FLAT // L, pos < 0)

    def body(st):
        j, pos = st
        chunk = flat_ref[pl.ds(j * L, L)]
        m = jnp.logical_and(chunk != xs_v, chunk != ys_v)
        c = lax.reduce_min(jnp.where(m, _iota(), _splat_i(L)), axes=(0,))
        pos2 = jnp.where(c < L, j * L + c, -1)
        return j + 1, pos2

    _, pos = lax.while_loop(cond, body, (jnp.int32(0), jnp.int32(-1)))
    pos = jnp.maximum(pos, 0)
    return _splat_at(flat_ref, pos)


def _sc_body(verts_hbm, faces_hbm, eps_hbm, out_hbm,
             sh_table, sh_pres, sh_xyz,
             f0slice, presidx1, myslice, verts_v,
             keyidx, v0s, v1s, cntbuf, presfull, xyzbuf,
             flat0_v, flat1_v, zeros_f, ones_f, ones_i, zeros_i, eps_v, acc_v,
             sem, sem1, sem2):
    cid = lax.axis_index("c")
    sid = lax.axis_index("s")

    @pl.when(sid == 0)
    def _():
        pltpu.async_copy(faces_hbm.at[pl.ds(0, FLAT)],
                         flat0_v.at[pl.ds(0, FLAT)], sem2)
        pltpu.async_copy(faces_hbm.at[pl.ds(FLAT, FLAT)],
                         flat1_v.at[pl.ds(0, FLAT)], sem2)

    base0 = sid * WS
    stage = []
    for k in range(3):
        stage.append(pltpu.async_copy(
            faces_hbm.at[pl.ds(0 * FLAT + base0 + k * FS, FS)], f0slice.at[k], sem))
        stage.append(pltpu.async_copy(
            faces_hbm.at[pl.ds(1 * FLAT + base0 + k * FS, FS)], presidx1.at[k], sem))
    stage.append(pltpu.async_copy(
        faces_hbm.at[pl.ds(cid * FLAT + base0, WS)], myslice, sem))
    stage.append(pltpu.async_copy(verts_hbm.at[cid], verts_v, sem))
    stage.append(pltpu.async_copy(eps_hbm, eps_v, sem))

    zf = jnp.zeros((L,), jnp.float32)
    zi = _splat_i(0)
    oi = _splat_i(1)
    of = jnp.full((L,), 1.0, jnp.float32)

    def fill_body(j, _):
        zeros_f[pl.ds(j * L, L)] = zf
        ones_f[pl.ds(j * L, L)] = of
        ones_i[pl.ds(j * L, L)] = oi
        return 0

    lax.fori_loop(0, FS // L, fill_body, 0)
    for j in range(2 * NV // NS // L):
        zeros_i[pl.ds(j * L, L)] = zi

    zp = pltpu.async_copy(
        zeros_i, sh_pres.at[pl.ds(sid * (2 * NV // NS), 2 * NV // NS)], sem1)
    for cp in stage:
        cp.wait()

    for k in range(3):
        def shift_body(j, _, k=k):
            presidx1[k, pl.ds(j * L, L)] = presidx1[k, pl.ds(j * L, L)] + NV
            return 0

        lax.fori_loop(0, FS // L, shift_body, 0)

    for h in range(2):
        def pa_body(t, _, h=h):
            jj = _iota() + t * L
            base = jj * 3 + h
            a = plsc.load_gather(myslice, [base])
            b = plsc.load_gather(myslice, [base + 1])
            v0 = jnp.minimum(a, b)
            v1 = jnp.maximum(a, b)
            v0s[pl.ds(h * FS + t * L, L)] = v0
            v1s[pl.ds(h * FS + t * L, L)] = v1
            keyidx[h, pl.ds(t * L, L)] = v0 * NV + v1
            return 0

        lax.fori_loop(0, FS // L, pa_body, 0)

    zk = [pltpu.async_copy(zeros_f, sh_table.at[keyidx.at[j]], sem1)
          for j in range(2)]

    @pl.when(sid == 0)
    def _():
        pltpu.make_async_copy(faces_hbm.at[pl.ds(0, FLAT)],
                              flat0_v.at[pl.ds(0, FLAT)], sem2).wait()
        pltpu.make_async_copy(faces_hbm.at[pl.ds(FLAT, FLAT)],
                              flat1_v.at[pl.ds(0, FLAT)], sem2).wait()
        vec = _splat_i(0)
        lanes = _iota()
        for slot, flat_ref in enumerate((flat0_v, flat1_v)):
            xs = _splat_at(flat_ref, 0)
            ys = _first_not_in(flat_ref, xs, xs)
            zs = _first_not_in(flat_ref, xs, ys)
            vec = jnp.where(lanes == 3 * slot, xs, vec)
            vec = jnp.where(lanes == 3 * slot + 1, ys, vec)
            vec = jnp.where(lanes == 3 * slot + 2, zs, vec)
        xyzbuf[...] = vec
        pltpu.sync_copy(xyzbuf, sh_xyz)

    zp.wait()
    for cp in zk:
        cp.wait()
    plsc.subcore_barrier()

    acc_cps = []
    for k in range(3):
        acc_cps.append(pltpu.async_copy(ones_i, sh_pres.at[f0slice.at[k]], sem))
        acc_cps.append(pltpu.async_copy(ones_i, sh_pres.at[presidx1.at[k]], sem))
    for j in range(2):
        acc_cps.append(pltpu.async_copy(ones_f, sh_table.at[keyidx.at[j]],
                                        sem, add=True))
    for cp in acc_cps:
        cp.wait()

    plsc.subcore_barrier()

    rd = [pltpu.async_copy(sh_pres, presfull, sem),
          pltpu.async_copy(sh_xyz, xyzbuf, sem)]
    rd += [pltpu.async_copy(sh_table.at[keyidx.at[j]],
                            cntbuf.at[pl.ds(j * FS, FS)], sem)
           for j in range(2)]
    for cp in rd:
        cp.wait()

    xv = xyzbuf[...]
    x0 = jnp.full((L,), xv[0], jnp.int32)
    y0 = jnp.full((L,), xv[1], jnp.int32)
    z0 = jnp.full((L,), xv[2], jnp.int32)
    x1 = jnp.full((L,), xv[3], jnp.int32)
    y1 = jnp.full((L,), xv[4], jnp.int32)
    z1 = jnp.full((L,), xv[5], jnp.int32)
    eps = eps_v[...]

    def pc_body(tt, acc):
        v0 = v0s[pl.ds(tt * L, L)]
        v1 = v1s[pl.ds(tt * L, L)]
        cnt = cntbuf[pl.ds(tt * L, L)]
        w = 1.0 / cnt

        m0 = jnp.logical_and(plsc.load_gather(presfull, [v0]) > 0,
                             plsc.load_gather(presfull, [v1]) > 0)
        m1 = jnp.logical_and(plsc.load_gather(presfull, [v0 + NV]) > 0,
                             plsc.load_gather(presfull, [v1 + NV]) > 0)
        val0 = jnp.where(jnp.logical_and(v0 != x0, v1 != x0), x0,
                         jnp.where(jnp.logical_and(v0 != y0, v1 != y0), y0, z0))
        val1 = jnp.where(jnp.logical_and(v0 != x1, v1 != x1), x1,
                         jnp.where(jnp.logical_and(v0 != y1, v1 != y1), y1, z1))
        v2 = jnp.where(m0, val0, jnp.where(m1, val1, 0))
        v3 = jnp.where(jnp.logical_and(m0, m1), val1, 0)

        def vert(vidx):
            b3 = vidx * 3
            return (plsc.load_gather(verts_v, [b3]),
                    plsc.load_gather(verts_v, [b3 + 1]),
                    plsc.load_gather(verts_v, [b3 + 2]))

        p0 = vert(v0)
        p1 = vert(v1)
        p2 = vert(v2)
        p3 = vert(v3)
        a1 = tuple(p1[k] - p0[k] for k in range(3))
        b1 = tuple(p2[k] - p0[k] for k in range(3))
        b2 = tuple(p3[k] - p0[k] for k in range(3))

        def dot3(u, v):
            return u[0] * v[0] + u[1] * v[1] + u[2] * v[2]

        a1l2 = dot3(a1, a1)
        b1l2 = dot3(b1, b1)
        b2l2 = dot3(b2, b2)
        ab1 = dot3(a1, b1)
        ab2 = dot3(a1, b2)
        a1l1 = _sqrt(a1l2 + eps)
        b1l1 = _sqrt(b1l2 + eps)
        b2l1 = _sqrt(b2l2 + eps)
        cos1 = ab1 / (a1l1 * b1l1 + eps)
        sin1 = _sqrt(1.0 - cos1 * cos1 + eps)
        cos2 = ab2 / (a1l1 * b2l1 + eps)
        sin2 = _sqrt(1.0 - cos2 * cos2 + eps)
        r1 = ab1 / (a1l2 + eps)
        r2 = ab2 / (a1l2 + eps)
        cb1 = tuple(b1[k] - a1[k] * r1 for k in range(3))
        cb2 = tuple(b2[k] - a1[k] * r2 for k in range(3))
        cosf = dot3(cb1, cb2) / (b1l1 * sin1 * b2l1 * sin2 + eps)
        term = (cosf + 1.0) * (cosf + 1.0)
        return acc + term * w

    acc = lax.fori_loop(0, ES // L, pc_body, jnp.zeros((L,), jnp.float32))
    acc_v[...] = acc
    pltpu.sync_copy(acc_v, out_hbm.at[cid * NS + sid])


def kernel(vertices, faces, eps):
    verts2d = vertices.reshape(2, 3 * NV).astype(jnp.float32)
    faces1d = faces.reshape(2 * FLAT).astype(jnp.int32)
    eps16 = jnp.full((L,), eps, jnp.float32)

    mesh = plsc.VectorSubcoreMesh(core_axis_name="c", subcore_axis_name="s")
    run = pl.kernel(
        _sc_body,
        mesh=mesh,
        out_type=jax.ShapeDtypeStruct((2 * NS, L), jnp.float32),
        compiler_params=pltpu.CompilerParams(needs_layout_passes=False),
        scratch_types=[
            pltpu.VMEM_SHARED((TBL,), jnp.float32),
            pltpu.VMEM_SHARED((2 * NV,), jnp.int32),
            pltpu.VMEM_SHARED((L,), jnp.int32),
            pltpu.VMEM((3, FS), jnp.int32),
            pltpu.VMEM((3, FS), jnp.int32),
            pltpu.VMEM((WS,), jnp.int32),
            pltpu.VMEM((3 * NV,), jnp.float32),
            pltpu.VMEM((2, FS), jnp.int32),
            pltpu.VMEM((ES,), jnp.int32),
            pltpu.VMEM((ES,), jnp.int32),
            pltpu.VMEM((ES,), jnp.float32),
            pltpu.VMEM((2 * NV,), jnp.int32),
            pltpu.VMEM((L,), jnp.int32),
            pltpu.VMEM((FLAT + L,), jnp.int32),
            pltpu.VMEM((FLAT + L,), jnp.int32),
            pltpu.VMEM((FS,), jnp.float32),
            pltpu.VMEM((FS,), jnp.float32),
            pltpu.VMEM((FS,), jnp.int32),
            pltpu.VMEM((2 * NV // NS,), jnp.int32),
            pltpu.VMEM((L,), jnp.float32),
            pltpu.VMEM((L,), jnp.float32),
            pltpu.SemaphoreType.DMA,
            pltpu.SemaphoreType.DMA,
            pltpu.SemaphoreType.DMA,
        ],
    )
    out = run(verts2d, faces1d, eps16)
    return jnp.sum(out)
